# Initial kernel scaffold; baseline (speedup 1.0000x reference)
import jax
import jax.numpy as jnp
from jax.experimental import pallas as pl


def kernel(context_words, central_words, negative_sampling, weight):
    raise NotImplementedError("write your pallas kernel here")


# speedup vs baseline: 2.4148x; 2.4148x over previous
import functools

import jax
import jax.numpy as jnp
from jax import lax
from jax.experimental import pallas as pl
from jax.experimental.pallas import tpu as pltpu
from jax.experimental.pallas import tpu_sc as plsc

V = 100000
D = 128
B = 1024
W = 20

NC = 2
NS = 16
NW = NC * NS
G = B * W + B
RPW = G // NW
CHUNK = 128

TV = 1024
NT = (V + TV - 1) // TV
GRID = W + NT


def _sc_gather(table, idx):
  mesh = plsc.VectorSubcoreMesh(core_axis_name="c", subcore_axis_name="s")

  @functools.partial(
      pl.kernel,
      out_type=jax.ShapeDtypeStruct((G, D), jnp.float32),
      mesh=mesh,
      scratch_types=[
          pltpu.VMEM((RPW,), jnp.int32),
          pltpu.VMEM((RPW, D), jnp.float32),
          pltpu.SemaphoreType.DMA,
      ],
  )
  def gather_kernel(table_hbm, idx_hbm, out_hbm, idx_v, rows_v, sem):
    wid = lax.axis_index("s") * NC + lax.axis_index("c")
    base = wid * RPW
    pltpu.sync_copy(idx_hbm.at[pl.ds(base, RPW)], idx_v)
    copies = []
    for k in range(0, RPW, CHUNK):
      sz = min(CHUNK, RPW - k)
      copies.append(
          pltpu.async_copy(
              table_hbm.at[idx_v.at[pl.ds(k, sz)]],
              rows_v.at[pl.ds(k, sz)],
              sem,
          ))
    for c in copies:
      c.wait()
    pltpu.sync_copy(rows_v, out_hbm.at[pl.ds(base, RPW)])

  return gather_kernel(table, idx)


def _fused_body(ctx_ref, cen_ref, w_ref, loss_ref, x_ref, acc_ref):
  i = pl.program_id(0)

  @pl.when(i == 0)
  def _init():
    x_ref[...] = jnp.zeros_like(x_ref)
    acc_ref[...] = jnp.zeros_like(acc_ref)

  @pl.when(i < W)
  def _accum_x():
    rows = ctx_ref[0]
    ss = jnp.sum(rows * rows, axis=1, keepdims=True)
    inv = 1.0 / jnp.maximum(jnp.sqrt(ss), 1e-12)
    x_ref[...] += rows * inv

  @pl.when(i >= W)
  def _vocab_tile():
    w = w_ref[...]
    ss = jnp.sum(w * w, axis=1, keepdims=True)
    inv = 1.0 / jnp.maximum(jnp.sqrt(ss), 1e-12)
    wn = (w * inv).astype(jnp.bfloat16)
    xb = x_ref[...].astype(jnp.bfloat16)
    s = lax.dot_general(xb, wn, (((1,), (1,)), ((), ())),
                        preferred_element_type=jnp.float32)
    es = jnp.exp(s)
    col = (i - W) * TV + lax.broadcasted_iota(jnp.int32, (1, TV), 1)
    es = jnp.where(col < V, es, 0.0)
    acc_ref[...] += jnp.sum(es, axis=1, keepdims=True)

  @pl.when(i == GRID - 1)
  def _epilogue():
    cen = cen_ref[...]
    ss = jnp.sum(cen * cen, axis=1, keepdims=True)
    inv = 1.0 / jnp.maximum(jnp.sqrt(ss), 1e-12)
    tgt = jnp.sum(x_ref[...] * (cen * inv), axis=1, keepdims=True)
    lvec = jnp.log(acc_ref[...]) - tgt
    loss_ref[...] = jnp.sum(lvec, axis=0, keepdims=True) / B


def kernel(context_words, central_words, negative_sampling, weight):
  del negative_sampling
  idx = jnp.concatenate(
      [context_words.T.reshape(-1), central_words.reshape(-1)]
  ).astype(jnp.int32)
  gathered = _sc_gather(weight, idx)
  ctx = gathered[: B * W].reshape(W, B, D)
  cen = gathered[B * W :]

  loss = pl.pallas_call(
      _fused_body,
      grid=(GRID,),
      in_specs=[
          pl.BlockSpec((1, B, D), lambda i: (jnp.minimum(i, W - 1), 0, 0)),
          pl.BlockSpec((B, D), lambda i: (0, 0)),
          pl.BlockSpec((TV, D), lambda i: (jnp.maximum(i - W, 0), 0)),
      ],
      out_specs=pl.BlockSpec((1, 1), lambda i: (0, 0)),
      out_shape=jax.ShapeDtypeStruct((1, 1), jnp.float32),
      scratch_shapes=[
          pltpu.VMEM((B, D), jnp.float32),
          pltpu.VMEM((B, 1), jnp.float32),
      ],
  )(ctx, cen, weight)
  return loss[0, 0]


# speedup vs baseline: 2.5178x; 1.0426x over previous
import functools

import jax
import jax.numpy as jnp
from jax import lax
from jax.experimental import pallas as pl
from jax.experimental.pallas import tpu as pltpu
from jax.experimental.pallas import tpu_sc as plsc

V = 100000
D = 128
B = 1024
W = 20

NC = 2
NS = 16
NW = NC * NS
G = B * W + B
RPW = G // NW
CHUNK = 128

TV = 1024
NT = (V + TV - 1) // TV
GRID = W + NT


def _sc_gather(table, idx):
  mesh = plsc.VectorSubcoreMesh(core_axis_name="c", subcore_axis_name="s")

  @functools.partial(
      pl.kernel,
      out_type=jax.ShapeDtypeStruct((G, D), jnp.float32),
      mesh=mesh,
      scratch_types=[
          pltpu.VMEM((RPW,), jnp.int32),
          pltpu.VMEM((RPW, D), jnp.float32),
          pltpu.SemaphoreType.DMA,
      ],
  )
  def gather_kernel(table_hbm, idx_hbm, out_hbm, idx_v, rows_v, sem):
    wid = lax.axis_index("s") * NC + lax.axis_index("c")
    base = wid * RPW
    pltpu.sync_copy(idx_hbm.at[pl.ds(base, RPW)], idx_v)
    copies = []
    for k in range(0, RPW, CHUNK):
      sz = min(CHUNK, RPW - k)
      copies.append(
          pltpu.async_copy(
              table_hbm.at[idx_v.at[pl.ds(k, sz)]],
              rows_v.at[pl.ds(k, sz)],
              sem,
          ))
    for c in copies:
      c.wait()
    pltpu.sync_copy(rows_v, out_hbm.at[pl.ds(base, RPW)])

  return gather_kernel(table, idx)


LOG2E = 1.4426950408889634
NPAD = NT * TV - V


def _fused_body(ctx_ref, cen_ref, w_ref, loss_ref, x_ref, xb_ref, acc_ref):
  i = pl.program_id(0)

  @pl.when(i == 0)
  def _init():
    x_ref[...] = jnp.zeros_like(x_ref)
    acc_ref[...] = jnp.zeros_like(acc_ref)

  @pl.when(i < W)
  def _accum_x():
    rows = ctx_ref[0]
    ss = jnp.sum(rows * rows, axis=1, keepdims=True)
    inv = 1.0 / jnp.maximum(jnp.sqrt(ss), 1e-12)
    x_ref[...] += rows * inv

  @pl.when(i == W - 1)
  def _freeze_x():
    xb_ref[...] = x_ref[...].astype(jnp.bfloat16)

  @pl.when(i >= W)
  def _vocab_tile():
    w = w_ref[...]
    row = (i - W) * TV + lax.broadcasted_iota(jnp.int32, (TV, 1), 0)
    wz = jnp.where(row < V, w, 0.0)
    ss = jnp.sum(wz * wz, axis=1, keepdims=True)
    inv = LOG2E / jnp.maximum(jnp.sqrt(ss), 1e-12)
    wn = (wz * inv).astype(jnp.bfloat16)
    s = lax.dot_general(xb_ref[...], wn, (((1,), (1,)), ((), ())),
                        preferred_element_type=jnp.float32)
    es = jnp.exp2(s)
    part = es[:, 0:D]
    for c in range(D, TV, D):
      part = part + es[:, c:c + D]
    acc_ref[...] += part

  @pl.when(i == GRID - 1)
  def _epilogue():
    cen = cen_ref[...]
    ss = jnp.sum(cen * cen, axis=1, keepdims=True)
    inv = 1.0 / jnp.maximum(jnp.sqrt(ss), 1e-12)
    tgt = jnp.sum(x_ref[...] * (cen * inv), axis=1, keepdims=True)
    z = jnp.sum(acc_ref[...], axis=1, keepdims=True) - NPAD
    lvec = jnp.log(z) - tgt
    loss_ref[...] = jnp.sum(lvec, axis=0, keepdims=True) / B


def kernel(context_words, central_words, negative_sampling, weight):
  del negative_sampling
  idx = jnp.concatenate(
      [context_words.T.reshape(-1), central_words.reshape(-1)]
  ).astype(jnp.int32)
  gathered = _sc_gather(weight, idx)
  ctx = gathered[: B * W].reshape(W, B, D)
  cen = gathered[B * W :]

  loss = pl.pallas_call(
      _fused_body,
      grid=(GRID,),
      in_specs=[
          pl.BlockSpec((1, B, D), lambda i: (jnp.minimum(i, W - 1), 0, 0)),
          pl.BlockSpec((B, D), lambda i: (0, 0)),
          pl.BlockSpec((TV, D), lambda i: (jnp.maximum(i - W, 0), 0)),
      ],
      out_specs=pl.BlockSpec((1, 1), lambda i: (0, 0)),
      out_shape=jax.ShapeDtypeStruct((1, 1), jnp.float32),
      scratch_shapes=[
          pltpu.VMEM((B, D), jnp.float32),
          pltpu.VMEM((B, D), jnp.bfloat16),
          pltpu.VMEM((B, D), jnp.float32),
      ],
  )(ctx, cen, weight)
  return loss[0, 0]


# speedup vs baseline: 2.7004x; 1.0725x over previous
import functools

import jax
import jax.numpy as jnp
from jax import lax
from jax.experimental import pallas as pl
from jax.experimental.pallas import tpu as pltpu
from jax.experimental.pallas import tpu_sc as plsc

V = 100000
D = 128
B = 1024
W = 20

NC = 2
NS = 16
NW = NC * NS
G = B * W + B
RPW = G // NW
CHUNK = 128

TV = 2048
NT = (V + TV - 1) // TV
GRID = W + NT


def _sc_gather(table, idx):
  mesh = plsc.VectorSubcoreMesh(core_axis_name="c", subcore_axis_name="s")

  @functools.partial(
      pl.kernel,
      out_type=jax.ShapeDtypeStruct((G, D), jnp.float32),
      mesh=mesh,
      scratch_types=[
          pltpu.VMEM((RPW,), jnp.int32),
          pltpu.VMEM((RPW, D), jnp.float32),
          pltpu.SemaphoreType.DMA,
      ],
  )
  def gather_kernel(table_hbm, idx_hbm, out_hbm, idx_v, rows_v, sem):
    wid = lax.axis_index("s") * NC + lax.axis_index("c")
    base = wid * RPW
    pltpu.sync_copy(idx_hbm.at[pl.ds(base, RPW)], idx_v)
    copies = []
    for k in range(0, RPW, CHUNK):
      sz = min(CHUNK, RPW - k)
      copies.append(
          pltpu.async_copy(
              table_hbm.at[idx_v.at[pl.ds(k, sz)]],
              rows_v.at[pl.ds(k, sz)],
              sem,
          ))
    for c in copies:
      c.wait()
    pltpu.sync_copy(rows_v, out_hbm.at[pl.ds(base, RPW)])

  return gather_kernel(table, idx)


LOG2E = 1.4426950408889634
NPAD = NT * TV - V


def _fused_body(ctx_ref, cen_ref, w_ref, loss_ref, x_ref, xb_ref, acc_ref):
  i = pl.program_id(0)

  @pl.when(i == 0)
  def _init():
    x_ref[...] = jnp.zeros_like(x_ref)
    acc_ref[...] = jnp.zeros_like(acc_ref)

  @pl.when(i < W)
  def _accum_x():
    rows = ctx_ref[0]
    ss = jnp.sum(rows * rows, axis=1, keepdims=True)
    inv = 1.0 / jnp.maximum(jnp.sqrt(ss), 1e-12)
    x_ref[...] += rows * inv

  @pl.when(i == W - 1)
  def _freeze_x():
    xb_ref[...] = x_ref[...].astype(jnp.bfloat16)

  @pl.when(i >= W)
  def _vocab_tile():
    w = w_ref[...]
    row = (i - W) * TV + lax.broadcasted_iota(jnp.int32, (TV, 1), 0)
    wz = jnp.where(row < V, w, 0.0)
    ss = jnp.sum(wz * wz, axis=1, keepdims=True)
    inv = LOG2E / jnp.maximum(jnp.sqrt(ss), 1e-12)
    wn = (wz * inv).astype(jnp.bfloat16)
    s = lax.dot_general(xb_ref[...], wn, (((1,), (1,)), ((), ())),
                        preferred_element_type=jnp.float32)
    es = jnp.exp2(s)
    part = es[:, 0:D]
    for c in range(D, TV, D):
      part = part + es[:, c:c + D]
    acc_ref[...] += part

  @pl.when(i == GRID - 1)
  def _epilogue():
    cen = cen_ref[...]
    ss = jnp.sum(cen * cen, axis=1, keepdims=True)
    inv = 1.0 / jnp.maximum(jnp.sqrt(ss), 1e-12)
    tgt = jnp.sum(x_ref[...] * (cen * inv), axis=1, keepdims=True)
    z = jnp.sum(acc_ref[...], axis=1, keepdims=True) - NPAD
    lvec = jnp.log(z) - tgt
    loss_ref[...] = jnp.sum(lvec, axis=0, keepdims=True) / B


def kernel(context_words, central_words, negative_sampling, weight):
  del negative_sampling
  idx = jnp.concatenate(
      [context_words.T.reshape(-1), central_words.reshape(-1)]
  ).astype(jnp.int32)
  gathered = _sc_gather(weight, idx)
  ctx = gathered[: B * W].reshape(W, B, D)
  cen = gathered[B * W :]

  loss = pl.pallas_call(
      _fused_body,
      grid=(GRID,),
      in_specs=[
          pl.BlockSpec((1, B, D), lambda i: (jnp.minimum(i, W - 1), 0, 0)),
          pl.BlockSpec((B, D), lambda i: (0, 0)),
          pl.BlockSpec((TV, D), lambda i: (jnp.maximum(i - W, 0), 0)),
      ],
      out_specs=pl.BlockSpec((1, 1), lambda i: (0, 0)),
      out_shape=jax.ShapeDtypeStruct((1, 1), jnp.float32),
      scratch_shapes=[
          pltpu.VMEM((B, D), jnp.float32),
          pltpu.VMEM((B, D), jnp.bfloat16),
          pltpu.VMEM((B, D), jnp.float32),
      ],
  )(ctx, cen, weight)
  return loss[0, 0]


# speedup vs baseline: 2.7482x; 1.0177x over previous
import functools

import jax
import jax.numpy as jnp
from jax import lax
from jax.experimental import pallas as pl
from jax.experimental.pallas import tpu as pltpu
from jax.experimental.pallas import tpu_sc as plsc

V = 100000
D = 128
B = 1024
W = 20

NC = 2
NS = 16
NW = NC * NS
G = B * W + B
RPW = G // NW
CHUNK = 128

TV = 4096
NT = (V + TV - 1) // TV
GRID = W + NT


def _sc_gather(table, idx):
  mesh = plsc.VectorSubcoreMesh(core_axis_name="c", subcore_axis_name="s")

  @functools.partial(
      pl.kernel,
      out_type=jax.ShapeDtypeStruct((G, D), jnp.float32),
      mesh=mesh,
      scratch_types=[
          pltpu.VMEM((RPW,), jnp.int32),
          pltpu.VMEM((RPW, D), jnp.float32),
          pltpu.SemaphoreType.DMA,
      ],
  )
  def gather_kernel(table_hbm, idx_hbm, out_hbm, idx_v, rows_v, sem):
    wid = lax.axis_index("s") * NC + lax.axis_index("c")
    base = wid * RPW
    pltpu.sync_copy(idx_hbm.at[pl.ds(base, RPW)], idx_v)
    copies = []
    for k in range(0, RPW, CHUNK):
      sz = min(CHUNK, RPW - k)
      copies.append(
          pltpu.async_copy(
              table_hbm.at[idx_v.at[pl.ds(k, sz)]],
              rows_v.at[pl.ds(k, sz)],
              sem,
          ))
    for c in copies:
      c.wait()
    pltpu.sync_copy(rows_v, out_hbm.at[pl.ds(base, RPW)])

  return gather_kernel(table, idx)


LOG2E = 1.4426950408889634
NPAD = NT * TV - V


def _fused_body(ctx_ref, cen_ref, w_ref, loss_ref, x_ref, xb_ref, acc_ref):
  i = pl.program_id(0)

  @pl.when(i == 0)
  def _init():
    x_ref[...] = jnp.zeros_like(x_ref)
    acc_ref[...] = jnp.zeros_like(acc_ref)

  @pl.when(i < W)
  def _accum_x():
    rows = ctx_ref[0]
    ss = jnp.sum(rows * rows, axis=1, keepdims=True)
    inv = 1.0 / jnp.maximum(jnp.sqrt(ss), 1e-12)
    x_ref[...] += rows * inv

  @pl.when(i == W - 1)
  def _freeze_x():
    xb_ref[...] = x_ref[...].astype(jnp.bfloat16)

  @pl.when(i >= W)
  def _vocab_tile():
    w = w_ref[...]
    row = (i - W) * TV + lax.broadcasted_iota(jnp.int32, (TV, 1), 0)
    wz = jnp.where(row < V, w, 0.0)
    ss = jnp.sum(wz * wz, axis=1, keepdims=True)
    inv = LOG2E / jnp.maximum(jnp.sqrt(ss), 1e-12)
    wn = (wz * inv).astype(jnp.bfloat16)
    s = lax.dot_general(xb_ref[...], wn, (((1,), (1,)), ((), ())),
                        preferred_element_type=jnp.float32)
    es = jnp.exp2(s)
    part = es[:, 0:D]
    for c in range(D, TV, D):
      part = part + es[:, c:c + D]
    acc_ref[...] += part

  @pl.when(i == GRID - 1)
  def _epilogue():
    cen = cen_ref[...]
    ss = jnp.sum(cen * cen, axis=1, keepdims=True)
    inv = 1.0 / jnp.maximum(jnp.sqrt(ss), 1e-12)
    tgt = jnp.sum(x_ref[...] * (cen * inv), axis=1, keepdims=True)
    z = jnp.sum(acc_ref[...], axis=1, keepdims=True) - NPAD
    lvec = jnp.log(z) - tgt
    loss_ref[...] = jnp.sum(lvec, axis=0, keepdims=True) / B


def kernel(context_words, central_words, negative_sampling, weight):
  del negative_sampling
  idx = jnp.concatenate(
      [context_words.T.reshape(-1), central_words.reshape(-1)]
  ).astype(jnp.int32)
  gathered = _sc_gather(weight, idx)
  ctx = gathered[: B * W].reshape(W, B, D)
  cen = gathered[B * W :]

  loss = pl.pallas_call(
      _fused_body,
      grid=(GRID,),
      in_specs=[
          pl.BlockSpec((1, B, D), lambda i: (jnp.minimum(i, W - 1), 0, 0)),
          pl.BlockSpec((B, D), lambda i: (0, 0)),
          pl.BlockSpec((TV, D), lambda i: (jnp.maximum(i - W, 0), 0)),
      ],
      out_specs=pl.BlockSpec((1, 1), lambda i: (0, 0)),
      out_shape=jax.ShapeDtypeStruct((1, 1), jnp.float32),
      scratch_shapes=[
          pltpu.VMEM((B, D), jnp.float32),
          pltpu.VMEM((B, D), jnp.bfloat16),
          pltpu.VMEM((B, D), jnp.float32),
      ],
  )(ctx, cen, weight)
  return loss[0, 0]
